# baseline (speedup 1.0000x reference)
import jax
import jax.numpy as jnp
from jax import lax
from jax.experimental import pallas as pl
from jax.experimental.pallas import tpu as pltpu
from jax.experimental.pallas import tpu_sc as plsc

_B = 128
_LANES = 16


def _build_sc_call(n_atoms, interpret=False):
    NW = 16
    PER = -(-n_atoms // NW)
    PER = -(-PER // _LANES) * _LANES
    while (PER * 3) % 8:
        PER += _LANES
    LAST_BASE = (NW - 1) * PER
    LAST = n_atoms - LAST_BASE
    assert LAST > 0 and LAST % _LANES == 0

    mesh = plsc.VectorSubcoreMesh(
        core_axis_name="c", subcore_axis_name="s",
        num_cores=1, num_subcores=NW)

    def body(pos_hbm, bid_hbm, out_hbm, pos_v, bid_v, acc_v, idx_v, shared,
             sem1, sem2):
        wid = lax.axis_index("s")
        lane = lax.iota(jnp.int32, _LANES)
        is_last = wid == (NW - 1)
        base = wid * PER

        @pl.when(~is_last)
        def _stage_full():
            pltpu.async_copy(
                pos_hbm.at[pl.ds(base * 3, PER * 3)], pos_v, sem1)
            pltpu.async_copy(
                bid_hbm.at[pl.ds(base, PER)], bid_v, sem2)

        @pl.when(is_last)
        def _stage_tail():
            pltpu.async_copy(
                pos_hbm.at[pl.ds(LAST_BASE * 3, LAST * 3)],
                pos_v.at[pl.ds(0, LAST * 3)], sem1)
            pltpu.async_copy(
                bid_hbm.at[pl.ds(LAST_BASE, LAST)],
                bid_v.at[pl.ds(0, LAST)], sem2)

        for k in range(_B // _LANES):
            acc_v[pl.ds(k * _LANES, _LANES)] = jnp.zeros((_LANES,), jnp.float32)
            idx_v[pl.ds(k * _LANES, _LANES)] = lane + (k * _LANES)

        @pl.when(wid == 0)
        def _zero_shared():
            pltpu.sync_copy(acc_v, shared)

        plsc.subcore_barrier()

        @pl.when(~is_last)
        def _wait_full():
            pltpu.make_async_copy(
                pos_hbm.at[pl.ds(base * 3, PER * 3)], pos_v, sem1).wait()
            pltpu.make_async_copy(
                bid_hbm.at[pl.ds(base, PER)], bid_v, sem2).wait()

        @pl.when(is_last)
        def _wait_tail():
            pltpu.make_async_copy(
                pos_hbm.at[pl.ds(LAST_BASE * 3, LAST * 3)],
                pos_v.at[pl.ds(0, LAST * 3)], sem1).wait()
            pltpu.make_async_copy(
                bid_hbm.at[pl.ds(LAST_BASE, LAST)],
                bid_v.at[pl.ds(0, LAST)], sem2).wait()

        n_at = jnp.where(is_last, LAST, PER)
        nblocks = jnp.where(is_last, LAST // _LANES, PER // _LANES)

        @plsc.parallel_loop(0, nblocks, 1, unroll=4)
        def _block(j):
            a0 = j * _LANES
            bid = bid_v[pl.ds(a0, _LANES)]
            nxt = jnp.minimum(lane + (a0 + 1), n_at - 1)
            bidn = plsc.load_gather(bid_v, [nxt])
            f0 = lane * 3 + a0 * 3
            x = plsc.load_gather(pos_v, [f0])
            y = plsc.load_gather(pos_v, [f0 + 1])
            z = plsc.load_gather(pos_v, [f0 + 2])
            s = plsc.cumsum(x * x + y * y + z * z)
            neq = bid != bidn
            last = lane == (_LANES - 1)
            plsc.addupdate_scatter(acc_v, [bid], s, mask=neq | last)
            plsc.addupdate_scatter(acc_v, [bidn], -s, mask=neq & (~last))

        pltpu.sync_copy(acc_v, shared.at[idx_v], add=True)
        plsc.subcore_barrier()

        @pl.when(wid == 0)
        def _write_out():
            pltpu.sync_copy(shared, out_hbm)

    return pl.kernel(
        body,
        out_type=jax.ShapeDtypeStruct((_B,), jnp.float32),
        mesh=mesh,
        scratch_types=[
            pltpu.VMEM((PER * 3,), jnp.float32),
            pltpu.VMEM((PER,), jnp.int32),
            pltpu.VMEM((_B,), jnp.float32),
            pltpu.VMEM((_B,), jnp.int32),
            pltpu.VMEM_SHARED((_B,), jnp.float32),
            pltpu.SemaphoreType.DMA,
            pltpu.SemaphoreType.DMA,
        ],
        compiler_params=pltpu.CompilerParams(
            needs_layout_passes=False,
            disable_bounds_checks=True,
            disable_semaphore_checks=True,
        ),
        interpret=interpret,
    )


def kernel(positions, batch_idx, num_graphs):
    del num_graphs
    call = _build_sc_call(positions.shape[0])
    out = call(positions.reshape(-1), batch_idx.astype(jnp.int32))
    return out.reshape(_B, 1)


# speedup vs baseline: 3.7107x; 3.7107x over previous
import jax
import jax.numpy as jnp
from jax import lax
from jax.experimental import pallas as pl
from jax.experimental.pallas import tpu as pltpu
from jax.experimental.pallas import tpu_sc as plsc

_B = 128
_LANES = 16


def _build_sc_call(n_atoms, interpret=False):
    NW = 16
    PER = -(-n_atoms // NW)
    PER = -(-PER // _LANES) * _LANES
    while PER % 8:
        PER += _LANES
    LAST_BASE = (NW - 1) * PER
    LAST = n_atoms - LAST_BASE
    assert LAST > 0 and LAST % _LANES == 0
    assert n_atoms % 8 == 0

    mesh = plsc.VectorSubcoreMesh(
        core_axis_name="c", subcore_axis_name="s",
        num_cores=1, num_subcores=NW)

    def body(pos_hbm, bid_hbm, out_hbm, pos_v, bid_v, acc_v, idx_v, shared,
             sem1, sem2):
        wid = lax.axis_index("s")
        lane = lax.iota(jnp.int32, _LANES)
        is_last = wid == (NW - 1)
        base = wid * PER

        @pl.when(~is_last)
        def _stage_full():
            for c in range(3):
                pltpu.async_copy(
                    pos_hbm.at[pl.ds(c * n_atoms + base, PER)],
                    pos_v.at[pl.ds(c * PER, PER)], sem1)
            pltpu.async_copy(bid_hbm.at[pl.ds(base, PER)], bid_v, sem2)

        @pl.when(is_last)
        def _stage_tail():
            for c in range(3):
                pltpu.async_copy(
                    pos_hbm.at[pl.ds(c * n_atoms + LAST_BASE, LAST)],
                    pos_v.at[pl.ds(c * PER, LAST)], sem1)
            pltpu.async_copy(bid_hbm.at[pl.ds(LAST_BASE, LAST)],
                             bid_v.at[pl.ds(0, LAST)], sem2)

        for k in range(_B // _LANES):
            acc_v[pl.ds(k * _LANES, _LANES)] = jnp.zeros((_LANES,), jnp.float32)
            idx_v[pl.ds(k * _LANES, _LANES)] = lane + (k * _LANES)

        @pl.when(wid == 0)
        def _zero_shared():
            pltpu.sync_copy(acc_v, shared)

        plsc.subcore_barrier()

        @pl.when(~is_last)
        def _wait_full():
            for c in range(3):
                pltpu.make_async_copy(
                    pos_hbm.at[pl.ds(c * n_atoms + base, PER)],
                    pos_v.at[pl.ds(c * PER, PER)], sem1).wait()
            pltpu.make_async_copy(
                bid_hbm.at[pl.ds(base, PER)], bid_v, sem2).wait()

        @pl.when(is_last)
        def _wait_tail():
            for c in range(3):
                pltpu.make_async_copy(
                    pos_hbm.at[pl.ds(c * n_atoms + LAST_BASE, LAST)],
                    pos_v.at[pl.ds(c * PER, LAST)], sem1).wait()
            pltpu.make_async_copy(
                bid_hbm.at[pl.ds(LAST_BASE, LAST)],
                bid_v.at[pl.ds(0, LAST)], sem2).wait()

        n_at = jnp.where(is_last, LAST, PER)
        nblocks = jnp.where(is_last, LAST // _LANES, PER // _LANES)

        @plsc.parallel_loop(0, nblocks, 1, unroll=4)
        def _block(j):
            a0 = j * _LANES
            bid = bid_v[pl.ds(a0, _LANES)]
            nxt = jnp.minimum(lane + (a0 + 1), n_at - 1)
            bidn = plsc.load_gather(bid_v, [nxt])
            x = pos_v[pl.ds(a0, _LANES)]
            y = pos_v[pl.ds(PER + a0, _LANES)]
            z = pos_v[pl.ds(2 * PER + a0, _LANES)]
            s = plsc.cumsum(x * x + y * y + z * z)
            neq = bid != bidn
            last = lane == (_LANES - 1)
            plsc.addupdate_scatter(acc_v, [bid], s, mask=neq | last)
            plsc.addupdate_scatter(acc_v, [bidn], -s, mask=neq & (~last))

        pltpu.sync_copy(acc_v, shared.at[idx_v], add=True)
        plsc.subcore_barrier()

        @pl.when(wid == 0)
        def _write_out():
            pltpu.sync_copy(shared, out_hbm)

    return pl.kernel(
        body,
        out_type=jax.ShapeDtypeStruct((_B,), jnp.float32),
        mesh=mesh,
        scratch_types=[
            pltpu.VMEM((3 * PER,), jnp.float32),
            pltpu.VMEM((PER,), jnp.int32),
            pltpu.VMEM((_B,), jnp.float32),
            pltpu.VMEM((_B,), jnp.int32),
            pltpu.VMEM_SHARED((_B,), jnp.float32),
            pltpu.SemaphoreType.DMA,
            pltpu.SemaphoreType.DMA,
        ],
        compiler_params=pltpu.CompilerParams(
            needs_layout_passes=False,
            disable_bounds_checks=True,
            disable_semaphore_checks=True,
        ),
        interpret=interpret,
    )


def kernel(positions, batch_idx, num_graphs):
    del num_graphs
    call = _build_sc_call(positions.shape[0])
    out = call(positions.T.reshape(-1), batch_idx.astype(jnp.int32))
    return out.reshape(_B, 1)
